# baseline (speedup 1.0000x reference)
import jax
import jax.numpy as jnp
from jax.experimental import pallas as pl
import jax.experimental.pallas.tpu as pltpu

B, S, N = 128, 2048, 64
E = 8
TOPK = 2
BB = 16
NS = 2
SS = S // NS


def _gating_kernel(x_ref, ws_ref, wg_ref, bg_ref, gates_ref, load_ref,
                   acc_ref):
    i = pl.program_id(0)
    j = pl.program_id(1)

    xb = x_ref[...].reshape(BB * N, SS)

    x_lin = jax.lax.dot_general(
        ws_ref[...], xb,
        (((1,), (0,)), ((), ())),
        preferred_element_type=jnp.float32,
    )

    part = jax.lax.dot_general(
        x_lin, wg_ref[...],
        (((1,), (0,)), ((), ())),
        preferred_element_type=jnp.float32,
    )

    @pl.when(j == 0)
    def _init_acc():
        acc_ref[...] = part

    @pl.when(j != 0)
    def _add_acc():
        acc_ref[...] += part

    @pl.when(j == NS - 1)
    def _finish():
        logits = acc_ref[...] + bg_ref[...]

        idx = jax.lax.broadcasted_iota(jnp.int32, (BB, E), 1)
        m1 = jnp.max(logits, axis=1, keepdims=True)
        i1 = jnp.min(jnp.where(logits == m1, idx, E), axis=1, keepdims=True)
        masked = jnp.where(idx == i1, -jnp.inf, logits)
        m2 = jnp.max(masked, axis=1, keepdims=True)
        i2 = jnp.min(jnp.where(masked == m2, idx, E), axis=1, keepdims=True)

        t = jnp.exp(m2 - m1)
        denom = 1.0 + t
        g1 = 1.0 / denom
        g2 = t / denom

        gates = jnp.where(idx == i1, g1, jnp.where(idx == i2, g2, 0.0))
        gates_ref[...] = gates

        partial = jnp.sum((gates > 0.0).astype(jnp.int32), axis=0,
                          keepdims=True)

        @pl.when(i == 0)
        def _init_load():
            load_ref[...] = partial

        @pl.when(i != 0)
        def _acc_load():
            load_ref[...] += partial


@jax.jit
def kernel(x, W_start, b_start, W_gate, b_gate):
    xt = x.transpose(0, 2, 1)
    w = W_start.reshape(N)
    ws = (jnp.eye(BB, dtype=jnp.float32)[:, :, None]
          * w[None, None, :]).reshape(BB, BB * N)
    bg_eff = b_gate + b_start[0] * jnp.sum(W_gate, axis=0)

    gates, load = pl.pallas_call(
        _gating_kernel,
        grid=(B // BB, NS),
        in_specs=[
            pl.BlockSpec((BB, N, SS), lambda i, j: (i, 0, j)),
            pl.BlockSpec((BB, BB * N), lambda i, j: (0, 0)),
            pl.BlockSpec((SS, E), lambda i, j: (j, 0)),
            pl.BlockSpec((E,), lambda i, j: (0,)),
        ],
        out_specs=[
            pl.BlockSpec((BB, E), lambda i, j: (i, 0)),
            pl.BlockSpec((1, E), lambda i, j: (0, 0)),
        ],
        out_shape=[
            jax.ShapeDtypeStruct((B, E), jnp.float32),
            jax.ShapeDtypeStruct((1, E), jnp.int32),
        ],
        scratch_shapes=[pltpu.VMEM((BB, E), jnp.float32)],
        compiler_params=pltpu.CompilerParams(
            dimension_semantics=("arbitrary", "arbitrary"),
        ),
    )(xt, ws, W_gate, bg_eff)
    return gates, load.reshape(E)


# speedup vs baseline: 1.2514x; 1.2514x over previous
import jax
import jax.numpy as jnp
from jax.experimental import pallas as pl
import jax.experimental.pallas.tpu as pltpu

B, S, N = 128, 2048, 64
E = 8
TOPK = 2
BB = 16


def _gating_kernel(x_ref, ws_ref, wg_ref, bg_ref, gates_ref, load_ref):
    i = pl.program_id(0)

    xb = x_ref[...].reshape(BB * N, S)

    x_lin = jax.lax.dot_general(
        ws_ref[...], xb,
        (((1,), (0,)), ((), ())),
        preferred_element_type=jnp.float32,
    )

    logits = jax.lax.dot_general(
        x_lin, wg_ref[...],
        (((1,), (0,)), ((), ())),
        preferred_element_type=jnp.float32,
    ) + bg_ref[...]

    idx = jax.lax.broadcasted_iota(jnp.int32, (BB, E), 1)
    m1 = jnp.max(logits, axis=1, keepdims=True)
    i1 = jnp.min(jnp.where(logits == m1, idx, E), axis=1, keepdims=True)
    masked = jnp.where(idx == i1, -jnp.inf, logits)
    m2 = jnp.max(masked, axis=1, keepdims=True)
    i2 = jnp.min(jnp.where(masked == m2, idx, E), axis=1, keepdims=True)

    t = jnp.exp(m2 - m1)
    denom = 1.0 + t
    g1 = 1.0 / denom
    g2 = t / denom

    gates = jnp.where(idx == i1, g1, jnp.where(idx == i2, g2, 0.0))
    gates_ref[...] = gates

    partial = jnp.sum((gates > 0.0).astype(jnp.int32), axis=0,
                      keepdims=True)

    @pl.when(i == 0)
    def _init_load():
        load_ref[...] = partial

    @pl.when(i != 0)
    def _acc_load():
        load_ref[...] += partial


@jax.jit
def kernel(x, W_start, b_start, W_gate, b_gate):
    xt = x.transpose(0, 2, 1)
    w = W_start.reshape(N)
    ws = (jnp.eye(BB, dtype=jnp.float32)[:, :, None]
          * w[None, None, :]).reshape(BB, BB * N)
    bg_eff = b_gate + b_start[0] * jnp.sum(W_gate, axis=0)

    gates, load = pl.pallas_call(
        _gating_kernel,
        grid=(B // BB,),
        in_specs=[
            pl.BlockSpec((BB, N, S), lambda i: (i, 0, 0)),
            pl.BlockSpec((BB, BB * N), lambda i: (0, 0)),
            pl.BlockSpec((S, E), lambda i: (0, 0)),
            pl.BlockSpec((E,), lambda i: (0,)),
        ],
        out_specs=[
            pl.BlockSpec((BB, E), lambda i: (i, 0)),
            pl.BlockSpec((1, E), lambda i: (0, 0)),
        ],
        out_shape=[
            jax.ShapeDtypeStruct((B, E), jnp.float32),
            jax.ShapeDtypeStruct((1, E), jnp.int32),
        ],
        compiler_params=pltpu.CompilerParams(
            dimension_semantics=("arbitrary",),
        ),
    )(xt, ws, W_gate, bg_eff)
    return gates, load.reshape(E)
